# Initial kernel scaffold; baseline (speedup 1.0000x reference)
import jax
import jax.numpy as jnp
from jax.experimental import pallas as pl


def kernel(x, ln_g, ln_b, spec_Wr, spec_br, spec_W1, spec_b1, spec_W2, spec_b2, sh_Wr, sh_br, sh_W1, sh_b1, sh_W2, sh_b2):
    raise NotImplementedError("write your pallas kernel here")


# speedup vs baseline: 3.1923x; 3.1923x over previous
import functools

import jax
import jax.numpy as jnp
from jax.experimental import pallas as pl
from jax.experimental.pallas import tpu as pltpu

DIM = 1024
HID = 2048
ES = 8
EH = 2
K = 2
T = 2048

ROUTER_TILE = 512
TOK_TILE = 512


def _router_body(x_ref, g_ref, b_ref, swr_ref, sbr_ref, hwr_ref, hbr_ref,
                 xn_ref, sl_ref, sp_ref, ti_ref, tp_ref, hl_ref, hp_ref):
    x = x_ref[...]
    mu = jnp.mean(x, axis=-1, keepdims=True)
    var = jnp.mean((x - mu) ** 2, axis=-1, keepdims=True)
    xn = (x - mu) / jnp.sqrt(var + 1e-5) * g_ref[0] + b_ref[0]
    xn_ref[...] = xn

    sl = jnp.dot(xn, swr_ref[...], preferred_element_type=jnp.float32) + sbr_ref[0]
    sl_ref[...] = sl
    sp = jax.nn.softmax(sl, axis=-1)
    sp_ref[...] = sp
    i1 = jnp.argmax(sp, axis=-1).astype(jnp.int32)
    v1 = jnp.max(sp, axis=-1)
    lane = jax.lax.broadcasted_iota(jnp.int32, sp.shape, 1)
    masked = jnp.where(lane == i1[:, None], -jnp.inf, sp)
    i2 = jnp.argmax(masked, axis=-1).astype(jnp.int32)
    v2 = jnp.max(masked, axis=-1)
    ti_ref[...] = jnp.stack([i1, i2], axis=-1)
    tp_ref[...] = jnp.stack([v1, v2], axis=-1)

    hl = jnp.dot(xn, hwr_ref[...], preferred_element_type=jnp.float32) + hbr_ref[0]
    hl_ref[...] = hl
    hp_ref[...] = jax.nn.softmax(hl, axis=-1)


def _experts_body(xn_ref, ti_ref, tp_ref, hp_ref,
                  sw1_ref, sb1_ref, sw2_ref, sb2_ref,
                  hw1_ref, hb1_ref, hw2_ref, hb2_ref,
                  out_ref):
    e = pl.program_id(0)
    hh = pl.program_id(1)
    t = pl.program_id(2)
    xn = xn_ref[...]
    rows = pl.ds(t * TOK_TILE, TOK_TILE)
    first = jnp.logical_and(e == 0, hh == 0)

    def ffn(w1, b1, w2, b2):
        h = jnp.dot(xn, w1, preferred_element_type=jnp.float32) + b1
        h = 0.5 * h * (1.0 + jax.lax.erf(h * 0.7071067811865476))
        y = jnp.dot(h, w2, preferred_element_type=jnp.float32)
        return y + jnp.where(hh == 0, 1.0, 0.0) * b2

    @pl.when(e < ES)
    def _():
        y = ffn(sw1_ref[0], sb1_ref[0], sw2_ref[0], sb2_ref[0])
        ti = ti_ref[...]
        tp = tp_ref[...]
        w = (tp[:, 0:1] * (ti[:, 0:1] == e).astype(jnp.float32)
             + tp[:, 1:2] * (ti[:, 1:2] == e).astype(jnp.float32))
        contrib = w * y

        @pl.when(first)
        def _():
            out_ref[rows, :] = contrib

        @pl.when(jnp.logical_not(first))
        def _():
            out_ref[rows, :] += contrib

    @pl.when(e >= ES)
    def _():
        y = ffn(hw1_ref[0], hb1_ref[0], hw2_ref[0], hb2_ref[0])
        eh = e - ES
        hp = hp_ref[...]
        lane = jax.lax.broadcasted_iota(jnp.int32, hp.shape, 1)
        w = jnp.sum(jnp.where(lane == eh, hp, 0.0), axis=1, keepdims=True)
        out_ref[rows, :] += w * y


def kernel(x, ln_g, ln_b, spec_Wr, spec_br, spec_W1, spec_b1, spec_W2, spec_b2,
           sh_Wr, sh_br, sh_W1, sh_b1, sh_W2, sh_b2):
    B = x.shape[0]
    x2 = x.reshape(T, DIM)

    n_rt = T // ROUTER_TILE
    router = pl.pallas_call(
        _router_body,
        grid=(n_rt,),
        in_specs=[
            pl.BlockSpec((ROUTER_TILE, DIM), lambda t: (t, 0)),
            pl.BlockSpec((1, DIM), lambda t: (0, 0)),
            pl.BlockSpec((1, DIM), lambda t: (0, 0)),
            pl.BlockSpec((DIM, ES), lambda t: (0, 0)),
            pl.BlockSpec((1, ES), lambda t: (0, 0)),
            pl.BlockSpec((DIM, EH), lambda t: (0, 0)),
            pl.BlockSpec((1, EH), lambda t: (0, 0)),
        ],
        out_specs=[
            pl.BlockSpec((ROUTER_TILE, DIM), lambda t: (t, 0)),
            pl.BlockSpec((ROUTER_TILE, ES), lambda t: (t, 0)),
            pl.BlockSpec((ROUTER_TILE, ES), lambda t: (t, 0)),
            pl.BlockSpec((ROUTER_TILE, K), lambda t: (t, 0)),
            pl.BlockSpec((ROUTER_TILE, K), lambda t: (t, 0)),
            pl.BlockSpec((ROUTER_TILE, EH), lambda t: (t, 0)),
            pl.BlockSpec((ROUTER_TILE, EH), lambda t: (t, 0)),
        ],
        out_shape=[
            jax.ShapeDtypeStruct((T, DIM), jnp.float32),
            jax.ShapeDtypeStruct((T, ES), jnp.float32),
            jax.ShapeDtypeStruct((T, ES), jnp.float32),
            jax.ShapeDtypeStruct((T, K), jnp.int32),
            jax.ShapeDtypeStruct((T, K), jnp.float32),
            jax.ShapeDtypeStruct((T, EH), jnp.float32),
            jax.ShapeDtypeStruct((T, EH), jnp.float32),
        ],
    )
    xn, sl, sp, ti, tp, hl, hp = router(
        x2, ln_g.reshape(1, DIM), ln_b.reshape(1, DIM),
        spec_Wr, spec_br.reshape(1, ES), sh_Wr, sh_br.reshape(1, EH))

    n_tt = T // TOK_TILE
    HH = HID // 2

    def se(e, hh, t):
        return jnp.minimum(e, ES - 1)

    def he(e, hh, t):
        return jnp.clip(e - ES, 0, EH - 1)

    experts = pl.pallas_call(
        _experts_body,
        grid=(ES + EH, 2, n_tt),
        in_specs=[
            pl.BlockSpec((TOK_TILE, DIM), lambda e, hh, t: (t, 0)),
            pl.BlockSpec((TOK_TILE, K), lambda e, hh, t: (t, 0)),
            pl.BlockSpec((TOK_TILE, K), lambda e, hh, t: (t, 0)),
            pl.BlockSpec((TOK_TILE, EH), lambda e, hh, t: (t, 0)),
            pl.BlockSpec((1, DIM, HH), lambda e, hh, t: (se(e, hh, t), 0, hh)),
            pl.BlockSpec((1, 1, HH), lambda e, hh, t: (se(e, hh, t), 0, hh)),
            pl.BlockSpec((1, HH, DIM), lambda e, hh, t: (se(e, hh, t), hh, 0)),
            pl.BlockSpec((1, 1, DIM), lambda e, hh, t: (se(e, hh, t), 0, 0)),
            pl.BlockSpec((1, DIM, HH), lambda e, hh, t: (he(e, hh, t), 0, hh)),
            pl.BlockSpec((1, 1, HH), lambda e, hh, t: (he(e, hh, t), 0, hh)),
            pl.BlockSpec((1, HH, DIM), lambda e, hh, t: (he(e, hh, t), hh, 0)),
            pl.BlockSpec((1, 1, DIM), lambda e, hh, t: (he(e, hh, t), 0, 0)),
        ],
        out_specs=pl.BlockSpec((T, DIM), lambda e, hh, t: (0, 0)),
        out_shape=jax.ShapeDtypeStruct((T, DIM), jnp.float32),
    )
    out = experts(
        xn, ti, tp, hp,
        spec_W1, spec_b1.reshape(ES, 1, HID), spec_W2, spec_b2.reshape(ES, 1, DIM),
        sh_W1, sh_b1.reshape(EH, 1, HID), sh_W2, sh_b2.reshape(EH, 1, DIM))

    return (out.reshape(B, T, DIM), sl.reshape(B, T, ES), sp.reshape(B, T, ES),
            ti.reshape(B, T, K), tp.reshape(B, T, K), hl.reshape(B, T, EH),
            hp.reshape(B, T, EH))
